# baseline (speedup 1.0000x reference)
import functools
import math

import jax
import jax.numpy as jnp
from jax import lax
from jax.experimental import pallas as pl
from jax.experimental.pallas import tpu as pltpu
from jax.experimental.pallas import tpu_sc as plsc

N = 10000
B = 100
E = 320000
C = 128
DM = 256
L = 8
H = 4
DH = C // H
NPG = N // B
P = 128
NP = B * P
NSC = 2
NSUB = 16
NW = NSC * NSUB
EC = 256
NBUF = 2
CH = 80
KCH = 8
NBLK = CH // KCH
IR = 2 * KCH
EPAD = NSUB * EC * CH
ACC_H = NP // 2
ACC_R = ACC_H + P
ZR = ACC_R // NSUB
WR = ACC_H // NSUB
G = 10
RG = G * P


def _ln(x, g, b):
    mu = jnp.mean(x, axis=-1, keepdims=True)
    xc = x - mu
    var = jnp.mean(xc * xc, axis=-1, keepdims=True)
    return xc * lax.rsqrt(var + 1e-5) * g + b


def _dot(a, b):
    return jnp.dot(a, b, preferred_element_type=jnp.float32)


def _temb_body(t_ref, w1, b1, w2, b2, out_ref):
    tf = t_ref[...]
    io = lax.broadcasted_iota(jnp.int32, (1, C // 2), 1).astype(jnp.float32)
    freqs = jnp.exp(io * (-math.log(10000.0) / (C // 2)))
    a = tf * freqs
    temb = jnp.concatenate([jnp.sin(a), jnp.cos(a)], axis=1)
    hh = jnp.maximum(_dot(temb, w1[...]) + b1[...], 0.0)
    out_ref[...] = _dot(hh, w2[...]) + b2[...]


def _temb(t_f, w1, b1, w2, b2):
    return pl.pallas_call(
        _temb_body,
        out_shape=jax.ShapeDtypeStruct((B, DM), jnp.float32),
    )(t_f, w1, b1, w2, b2)


def _inproj_body(xt_ref, te_ref, w_ref, b_ref, out_ref):
    rows = [xt_ref[j * P:(j + 1) * P, :] + te_ref[0, j] for j in range(G)]
    xin = jnp.concatenate(rows, axis=0)
    out_ref[...] = _dot(xin, w_ref[...]) + b_ref[...]


def _inproj(x_tp, temb, w, b):
    return pl.pallas_call(
        _inproj_body,
        grid=(B // G,),
        in_specs=[
            pl.BlockSpec((RG, DM), lambda i: (i, 0)),
            pl.BlockSpec((1, G, DM), lambda i: (i, 0, 0)),
            pl.BlockSpec((DM, C), lambda i: (0, 0)),
            pl.BlockSpec((1, C), lambda i: (0, 0)),
        ],
        out_specs=pl.BlockSpec((RG, C), lambda i: (i, 0)),
        out_shape=jax.ShapeDtypeStruct((NP, C), jnp.float32),
    )(x_tp, temb, w, b)


def _layer_body(eps_ref, x_ref, agg_ref,
                gw1, gb1, gw2, gb2,
                wq, bq, wk, bk, wv, bv, wo, bo,
                fw1, fb1, fw2, fb2,
                g1, b1, g2, b2, g3, b3, out_ref):
    x = x_ref[...]
    agg = agg_ref[...]
    e1 = 1.0 + eps_ref[0]
    z = e1 * x + agg
    hg = jnp.maximum(_dot(z, gw1[...]) + gb1[...], 0.0)
    hg = _dot(hg, gw2[...]) + gb2[...]
    hg = _ln(hg + x, g1[...], b1[...])

    q = _dot(x, wq[...]) + bq[...]
    k = _dot(x, wk[...]) + bk[...]
    v = _dot(x, wv[...]) + bv[...]
    lane = lax.broadcasted_iota(jnp.int32, (P, C), 1)
    hmasks = [(lane // DH) == h for h in range(H)]
    smask = lax.broadcasted_iota(jnp.int32, (P, P), 1) >= NPG
    scale = 1.0 / math.sqrt(DH)
    outs = []
    for j in range(G):
        qj = q[j * P:(j + 1) * P, :]
        kj = k[j * P:(j + 1) * P, :]
        vj = v[j * P:(j + 1) * P, :]
        oj = None
        for h in range(H):
            kh = jnp.where(hmasks[h], kj, 0.0)
            s = lax.dot_general(qj, kh, (((1,), (1,)), ((), ())),
                                preferred_element_type=jnp.float32) * scale
            s = jnp.where(smask, -1e30, s)
            m = jnp.max(s, axis=-1, keepdims=True)
            p_ = jnp.exp(s - m)
            p_ = p_ / jnp.sum(p_, axis=-1, keepdims=True)
            vh = jnp.where(hmasks[h], vj, 0.0)
            contrib = _dot(p_, vh)
            oj = contrib if oj is None else oj + contrib
        outs.append(oj)
    o = jnp.concatenate(outs, axis=0)
    ha = _dot(o, wo[...]) + bo[...]
    ha = _ln(ha + x, g2[...], b2[...])

    out = hg + ha
    hid = jnp.maximum(_dot(out, fw1[...]) + fb1[...], 0.0)
    out = out + _dot(hid, fw2[...]) + fb2[...]
    out_ref[...] = _ln(out, g3[...], b3[...])


def _layer(eps, x, agg2, gw1, gb1, gw2, gb2, wq, bq, wk, bk, wv, bv, wo, bo,
           fw1, fb1, fw2, fb2, g1, b1, g2, b2, g3, b3):
    full = lambda r, c: pl.BlockSpec((r, c), lambda i: (0, 0))
    return pl.pallas_call(
        _layer_body,
        grid=(B // G,),
        in_specs=[
            pl.BlockSpec(memory_space=pltpu.SMEM),
            pl.BlockSpec((RG, C), lambda i: (i, 0)),
            pl.BlockSpec((RG, C), lambda i: (i, 0)),
            full(C, C), full(1, C), full(C, C), full(1, C),
            full(C, C), full(1, C), full(C, C), full(1, C),
            full(C, C), full(1, C), full(C, C), full(1, C),
            full(C, 2 * C), full(1, 2 * C), full(2 * C, C), full(1, C),
            full(1, C), full(1, C), full(1, C), full(1, C),
            full(1, C), full(1, C),
        ],
        out_specs=pl.BlockSpec((RG, C), lambda i: (i, 0)),
        out_shape=jax.ShapeDtypeStruct((NP, C), jnp.float32),
    )(eps, x, agg2, gw1, gb1, gw2, gb2, wq, bq, wk, bk, wv, bv, wo, bo,
      fw1, fb1, fw2, fb2, g1, b1, g2, b2, g3, b3)


def _outproj_body(x_ref, w_ref, b_ref, out_ref):
    out_ref[...] = _dot(x_ref[...], w_ref[...]) + b_ref[...]


def _outproj(x, w, b):
    return pl.pallas_call(
        _outproj_body,
        grid=(B // G,),
        in_specs=[
            pl.BlockSpec((RG, C), lambda i: (i, 0)),
            pl.BlockSpec((C, C), lambda i: (0, 0)),
            pl.BlockSpec((1, C), lambda i: (0, 0)),
        ],
        out_specs=pl.BlockSpec((RG, C), lambda i: (i, 0)),
        out_shape=jax.ShapeDtypeStruct((NP, C), jnp.float32),
    )(x, w, b)


def _sc_agg(x_p, src_i, dst_i, zrows):
    mesh = plsc.VectorSubcoreMesh(core_axis_name="c", subcore_axis_name="s")

    @functools.partial(
        pl.kernel,
        out_type=jax.ShapeDtypeStruct((NP, C), jnp.float32),
        mesh=mesh,
        scratch_types=[
            pltpu.VMEM((2 * KCH * EC,), jnp.int32),
            pltpu.VMEM((2 * IR, 128), jnp.int32),
            pltpu.VMEM((NBUF, EC, C), jnp.float32),
            pltpu.VMEM_SHARED((ACC_R, C), jnp.float32),
            [pltpu.SemaphoreType.DMA] * NBUF,
            [pltpu.SemaphoreType.DMA] * NBUF,
            pltpu.SemaphoreType.DMA,
        ],
    )
    def agg_kernel(x_hbm, src_hbm, dst_hbm, z_hbm, out_hbm,
                   src_v, dst_v, gbuf, acc, gsems, ssems, isem):
        cid = lax.axis_index("c")
        sid = lax.axis_index("s")
        pltpu.sync_copy(z_hbm, acc.at[pl.ds(sid * ZR, ZR)])
        pltpu.sync_copy(src_hbm.at[sid, pl.ds(0, 2 * KCH * EC)], src_v)
        pltpu.sync_copy(dst_hbm.at[cid, sid, pl.ds(0, 2 * IR)], dst_v)
        plsc.subcore_barrier()

        def srows(j):
            return pl.ds((j % (2 * KCH)) * EC, EC)

        def drow(j, k):
            return (2 * j + k) % (2 * IR)

        def fire(j, b):
            pltpu.async_copy(x_hbm.at[src_v.at[srows(j)]],
                             gbuf.at[b], gsems[b])

        def wait_g(j, b):
            pltpu.make_async_copy(x_hbm.at[src_v.at[srows(j)]],
                                  gbuf.at[b], gsems[b]).wait()

        def fire_s(j, b):
            for k in range(2):
                pltpu.async_copy(gbuf.at[b, pl.ds(128 * k, 128)],
                                 acc.at[dst_v.at[drow(j, k)]],
                                 ssems[b], add=True)

        def wait_s(j, b):
            for k in range(2):
                pltpu.make_async_copy(gbuf.at[b, pl.ds(128 * k, 128)],
                                      acc.at[dst_v.at[drow(j, k)]],
                                      ssems[b]).wait()

        fire(0, 0)

        def group(g, carry):
            for b in range(NBUF):
                j = g * NBUF + b
                bi = j // KCH
                wait_g(j, b)

                @pl.when(j + 1 < CH)
                def _():
                    fire(j + 1, (b + 1) % NBUF)

                @pl.when(j >= NBUF)
                def _():
                    wait_s(j - NBUF, b)

                fire_s(j, b)

                @pl.when(jnp.logical_and(j % KCH == 2,
                                         jnp.logical_and(bi >= 1,
                                                         bi + 1 < NBLK)))
                def _():
                    half = (bi + 1) % 2 * KCH
                    pltpu.async_copy(
                        src_hbm.at[sid, pl.ds((bi + 1) * KCH * EC, KCH * EC)],
                        src_v.at[pl.ds(half * EC, KCH * EC)], isem)
                    pltpu.async_copy(
                        dst_hbm.at[cid, sid, pl.ds((bi + 1) * IR, IR)],
                        dst_v.at[pl.ds(((bi + 1) % 2) * IR, IR)], isem)

                @pl.when(jnp.logical_and(j % KCH == 5,
                                         jnp.logical_and(bi >= 1,
                                                         bi + 1 < NBLK)))
                def _():
                    half = (bi + 1) % 2 * KCH
                    pltpu.make_async_copy(
                        src_hbm.at[sid, pl.ds((bi + 1) * KCH * EC, KCH * EC)],
                        src_v.at[pl.ds(half * EC, KCH * EC)], isem).wait()
                    pltpu.make_async_copy(
                        dst_hbm.at[cid, sid, pl.ds((bi + 1) * IR, IR)],
                        dst_v.at[pl.ds(((bi + 1) % 2) * IR, IR)], isem).wait()
            return carry

        lax.fori_loop(0, CH // NBUF, group, 0)
        for i in range(NBUF):
            wait_s(CH - NBUF + i, (CH - NBUF + i) % NBUF)
        plsc.subcore_barrier()
        pltpu.sync_copy(acc.at[pl.ds(sid * WR, WR)],
                        out_hbm.at[pl.ds(cid * ACC_H + sid * WR, WR)])

    return agg_kernel(x_p, src_i, dst_i, zrows)


def kernel(x_t, edge_index, batch, num_nodes, t, time_W1, time_b1, time_W2,
           time_b2, in_W, in_b, out_W, out_b, gin_eps, gin_W1, gin_b1,
           gin_W2, gin_b2, Wq, bq, Wk, bk, Wv, bv, Wo, bo, ffn_W1, ffn_b1,
           ffn_W2, ffn_b2, n1_g, n1_b, n2_g, n2_b, n3_g, n3_b):
    r = lambda a: a.reshape(1, -1)

    src = edge_index[0]
    dst = edge_index[1]
    srcp = (src // NPG) * P + (src % NPG)
    dstp = (dst // NPG) * P + (dst % NPG)
    pad = EPAD - E
    srcp = jnp.concatenate([srcp, jnp.zeros((pad,), jnp.int32)])
    dstp = jnp.concatenate([dstp, jnp.full((pad,), NP, jnp.int32)])
    dummy = ACC_H + (dstp % P)
    d0 = jnp.where(dstp < ACC_H, dstp, dummy)
    d1 = jnp.where((dstp >= ACC_H) & (dstp < NP), dstp - ACC_H, dummy)
    src_i = srcp.reshape(NSUB, CH * EC)
    dst_i = jnp.stack([d0, d1]).reshape(NSC, NSUB, 2 * CH, 128)
    zrows = jnp.zeros((ZR, C), jnp.float32)

    x_tp = (jnp.zeros((B, P, DM), jnp.float32)
            .at[:, :NPG, :].set(x_t.reshape(B, NPG, DM))
            .reshape(NP, DM))
    t_f = t.astype(jnp.float32).reshape(B, 1)

    temb = _temb(t_f, time_W1, r(time_b1), time_W2, r(time_b2))
    h = _inproj(x_tp, temb.reshape(B // G, G, DM), in_W, r(in_b))
    for l in range(L):
        agg2 = _sc_agg(h, src_i, dst_i, zrows)
        h = _layer(gin_eps[l:l + 1], h, agg2,
                   gin_W1[l], r(gin_b1[l]), gin_W2[l], r(gin_b2[l]),
                   Wq[l], r(bq[l]), Wk[l], r(bk[l]), Wv[l], r(bv[l]),
                   Wo[l], r(bo[l]),
                   ffn_W1[l], r(ffn_b1[l]), ffn_W2[l], r(ffn_b2[l]),
                   r(n1_g[l]), r(n1_b[l]), r(n2_g[l]), r(n2_b[l]),
                   r(n3_g[l]), r(n3_b[l]))
    outp = _outproj(h, out_W, r(out_b))
    return outp.reshape(B, P, C)[:, :NPG, :].reshape(N, C)


# speedup vs baseline: 1.0826x; 1.0826x over previous
import functools
import math

import jax
import jax.numpy as jnp
from jax import lax
from jax.experimental import pallas as pl
from jax.experimental.pallas import tpu as pltpu
from jax.experimental.pallas import tpu_sc as plsc

N = 10000
B = 100
E = 320000
C = 128
DM = 256
L = 8
H = 4
DH = C // H
NPG = N // B
P = 128
NP = B * P
NSC = 2
NSUB = 16
NW = NSC * NSUB
EC = 256
NBUF = 2
CHCAP = 80
KCH = 8
IR = 2 * KCH
ECAP = CHCAP * EC
ACC_H = NP // 2
ACC_R = ACC_H + P
ZR = ACC_R // NSUB
WR = ACC_H // NSUB
G = 10
RG = G * P


def _ln(x, g, b):
    mu = jnp.mean(x, axis=-1, keepdims=True)
    xc = x - mu
    var = jnp.mean(xc * xc, axis=-1, keepdims=True)
    return xc * lax.rsqrt(var + 1e-5) * g + b


def _dot(a, b):
    return jnp.dot(a, b, preferred_element_type=jnp.float32)


def _temb_body(t_ref, w1, b1, w2, b2, out_ref):
    tf = t_ref[...]
    io = lax.broadcasted_iota(jnp.int32, (1, C // 2), 1).astype(jnp.float32)
    freqs = jnp.exp(io * (-math.log(10000.0) / (C // 2)))
    a = tf * freqs
    temb = jnp.concatenate([jnp.sin(a), jnp.cos(a)], axis=1)
    hh = jnp.maximum(_dot(temb, w1[...]) + b1[...], 0.0)
    out_ref[...] = _dot(hh, w2[...]) + b2[...]


def _temb(t_f, w1, b1, w2, b2):
    return pl.pallas_call(
        _temb_body,
        out_shape=jax.ShapeDtypeStruct((B, DM), jnp.float32),
    )(t_f, w1, b1, w2, b2)


def _inproj_body(xt_ref, te_ref, w_ref, b_ref, out_ref):
    rows = [xt_ref[j * P:(j + 1) * P, :] + te_ref[0, j] for j in range(G)]
    xin = jnp.concatenate(rows, axis=0)
    out_ref[...] = _dot(xin, w_ref[...]) + b_ref[...]


def _inproj(x_tp, temb, w, b):
    return pl.pallas_call(
        _inproj_body,
        grid=(B // G,),
        in_specs=[
            pl.BlockSpec((RG, DM), lambda i: (i, 0)),
            pl.BlockSpec((1, G, DM), lambda i: (i, 0, 0)),
            pl.BlockSpec((DM, C), lambda i: (0, 0)),
            pl.BlockSpec((1, C), lambda i: (0, 0)),
        ],
        out_specs=pl.BlockSpec((RG, C), lambda i: (i, 0)),
        out_shape=jax.ShapeDtypeStruct((NP, C), jnp.float32),
    )(x_tp, temb, w, b)


def _layer_body(eps_ref, x_ref, agg_ref,
                gw1, gb1, gw2, gb2,
                wq, bq, wk, bk, wv, bv, wo, bo,
                fw1, fb1, fw2, fb2,
                g1, b1, g2, b2, g3, b3, out_ref):
    x = x_ref[...]
    agg = agg_ref[...]
    e1 = 1.0 + eps_ref[0]
    z = e1 * x + agg
    hg = jnp.maximum(_dot(z, gw1[...]) + gb1[...], 0.0)
    hg = _dot(hg, gw2[...]) + gb2[...]
    hg = _ln(hg + x, g1[...], b1[...])

    q = _dot(x, wq[...]) + bq[...]
    k = _dot(x, wk[...]) + bk[...]
    v = _dot(x, wv[...]) + bv[...]
    lane = lax.broadcasted_iota(jnp.int32, (P, C), 1)
    hmasks = [(lane // DH) == h for h in range(H)]
    smask = lax.broadcasted_iota(jnp.int32, (P, P), 1) >= NPG
    scale = 1.0 / math.sqrt(DH)
    outs = []
    for j in range(G):
        qj = q[j * P:(j + 1) * P, :]
        kj = k[j * P:(j + 1) * P, :]
        vj = v[j * P:(j + 1) * P, :]
        oj = None
        for h in range(H):
            kh = jnp.where(hmasks[h], kj, 0.0)
            s = lax.dot_general(qj, kh, (((1,), (1,)), ((), ())),
                                preferred_element_type=jnp.float32) * scale
            s = jnp.where(smask, -1e30, s)
            m = jnp.max(s, axis=-1, keepdims=True)
            p_ = jnp.exp(s - m)
            p_ = p_ / jnp.sum(p_, axis=-1, keepdims=True)
            vh = jnp.where(hmasks[h], vj, 0.0)
            contrib = _dot(p_, vh)
            oj = contrib if oj is None else oj + contrib
        outs.append(oj)
    o = jnp.concatenate(outs, axis=0)
    ha = _dot(o, wo[...]) + bo[...]
    ha = _ln(ha + x, g2[...], b2[...])

    out = hg + ha
    hid = jnp.maximum(_dot(out, fw1[...]) + fb1[...], 0.0)
    out = out + _dot(hid, fw2[...]) + fb2[...]
    out_ref[...] = _ln(out, g3[...], b3[...])


def _layer(eps, x, agg2, gw1, gb1, gw2, gb2, wq, bq, wk, bk, wv, bv, wo, bo,
           fw1, fb1, fw2, fb2, g1, b1, g2, b2, g3, b3):
    full = lambda r, c: pl.BlockSpec((r, c), lambda i: (0, 0))
    return pl.pallas_call(
        _layer_body,
        grid=(B // G,),
        in_specs=[
            pl.BlockSpec(memory_space=pltpu.SMEM),
            pl.BlockSpec((RG, C), lambda i: (i, 0)),
            pl.BlockSpec((RG, C), lambda i: (i, 0)),
            full(C, C), full(1, C), full(C, C), full(1, C),
            full(C, C), full(1, C), full(C, C), full(1, C),
            full(C, C), full(1, C), full(C, C), full(1, C),
            full(C, 2 * C), full(1, 2 * C), full(2 * C, C), full(1, C),
            full(1, C), full(1, C), full(1, C), full(1, C),
            full(1, C), full(1, C),
        ],
        out_specs=pl.BlockSpec((RG, C), lambda i: (i, 0)),
        out_shape=jax.ShapeDtypeStruct((NP, C), jnp.float32),
    )(eps, x, agg2, gw1, gb1, gw2, gb2, wq, bq, wk, bk, wv, bv, wo, bo,
      fw1, fb1, fw2, fb2, g1, b1, g2, b2, g3, b3)


def _outproj_body(x_ref, w_ref, b_ref, out_ref):
    out_ref[...] = _dot(x_ref[...], w_ref[...]) + b_ref[...]


def _outproj(x, w, b):
    return pl.pallas_call(
        _outproj_body,
        grid=(B // G,),
        in_specs=[
            pl.BlockSpec((RG, C), lambda i: (i, 0)),
            pl.BlockSpec((C, C), lambda i: (0, 0)),
            pl.BlockSpec((1, C), lambda i: (0, 0)),
        ],
        out_specs=pl.BlockSpec((RG, C), lambda i: (i, 0)),
        out_shape=jax.ShapeDtypeStruct((NP, C), jnp.float32),
    )(x, w, b)


def _sc_agg(x_p, src_i, dst_i, zrows, chn):
    mesh = plsc.VectorSubcoreMesh(core_axis_name="c", subcore_axis_name="s")

    @functools.partial(
        pl.kernel,
        out_type=jax.ShapeDtypeStruct((NP, C), jnp.float32),
        mesh=mesh,
        scratch_types=[
            pltpu.VMEM((2 * KCH * EC,), jnp.int32),
            pltpu.VMEM((2 * IR, 128), jnp.int32),
            pltpu.VMEM((NBUF, EC, C), jnp.float32),
            pltpu.VMEM((16,), jnp.int32),
            pltpu.VMEM_SHARED((ACC_R, C), jnp.float32),
            [pltpu.SemaphoreType.DMA] * NBUF,
            [pltpu.SemaphoreType.DMA] * NBUF,
            pltpu.SemaphoreType.DMA,
        ],
    )
    def agg_kernel(x_hbm, src_hbm, dst_hbm, z_hbm, chn_hbm, out_hbm,
                   src_v, dst_v, gbuf, cnt_v, acc, gsems, ssems, isem):
        cid = lax.axis_index("c")
        sid = lax.axis_index("s")
        pltpu.sync_copy(z_hbm, acc.at[pl.ds(sid * ZR, ZR)])
        pltpu.sync_copy(chn_hbm.at[cid, sid], cnt_v)
        pltpu.sync_copy(src_hbm.at[cid, sid, pl.ds(0, 2 * KCH * EC)], src_v)
        pltpu.sync_copy(dst_hbm.at[cid, sid, pl.ds(0, 2 * IR)], dst_v)
        nch = cnt_v[...][0]
        nblk = (nch + KCH - 1) // KCH
        plsc.subcore_barrier()

        def srows(j):
            return pl.ds((j % (2 * KCH)) * EC, EC)

        def drow(j, k):
            return (2 * j + k) % (2 * IR)

        def fire(j, b):
            pltpu.async_copy(x_hbm.at[src_v.at[srows(j)]],
                             gbuf.at[b], gsems[b])

        def wait_g(j, b):
            pltpu.make_async_copy(x_hbm.at[src_v.at[srows(j)]],
                                  gbuf.at[b], gsems[b]).wait()

        def fire_s(j, b):
            for k in range(2):
                pltpu.async_copy(gbuf.at[b, pl.ds(128 * k, 128)],
                                 acc.at[dst_v.at[drow(j, k)]],
                                 ssems[b], add=True)

        def wait_s(j, b):
            for k in range(2):
                pltpu.make_async_copy(gbuf.at[b, pl.ds(128 * k, 128)],
                                      acc.at[dst_v.at[drow(j, k)]],
                                      ssems[b]).wait()

        fire(0, 0)

        def group(g, carry):
            for b in range(NBUF):
                j = g * NBUF + b
                bi = j // KCH
                wait_g(j, b)

                @pl.when(j + 1 < nch)
                def _():
                    fire(j + 1, (b + 1) % NBUF)

                @pl.when(j >= NBUF)
                def _():
                    wait_s(j - NBUF, b)

                fire_s(j, b)

                @pl.when(jnp.logical_and(j % KCH == 2,
                                         jnp.logical_and(bi >= 1,
                                                         bi + 1 < nblk)))
                def _():
                    half = (bi + 1) % 2
                    pltpu.async_copy(
                        src_hbm.at[cid, sid,
                                   pl.ds((bi + 1) * KCH * EC, KCH * EC)],
                        src_v.at[pl.ds(half * KCH * EC, KCH * EC)], isem)
                    pltpu.async_copy(
                        dst_hbm.at[cid, sid, pl.ds((bi + 1) * IR, IR)],
                        dst_v.at[pl.ds(half * IR, IR)], isem)

                @pl.when(jnp.logical_and(j % KCH == 5,
                                         jnp.logical_and(bi >= 1,
                                                         bi + 1 < nblk)))
                def _():
                    half = (bi + 1) % 2
                    pltpu.make_async_copy(
                        src_hbm.at[cid, sid,
                                   pl.ds((bi + 1) * KCH * EC, KCH * EC)],
                        src_v.at[pl.ds(half * KCH * EC, KCH * EC)],
                        isem).wait()
                    pltpu.make_async_copy(
                        dst_hbm.at[cid, sid, pl.ds((bi + 1) * IR, IR)],
                        dst_v.at[pl.ds(half * IR, IR)], isem).wait()
            return carry

        lax.fori_loop(0, nch // NBUF, group, 0)
        wait_s(nch - 2, 0)
        wait_s(nch - 1, 1)
        plsc.subcore_barrier()
        pltpu.sync_copy(acc.at[pl.ds(sid * WR, WR)],
                        out_hbm.at[pl.ds(cid * ACC_H + sid * WR, WR)])

    return agg_kernel(x_p, src_i, dst_i, zrows, chn)


def kernel(x_t, edge_index, batch, num_nodes, t, time_W1, time_b1, time_W2,
           time_b2, in_W, in_b, out_W, out_b, gin_eps, gin_W1, gin_b1,
           gin_W2, gin_b2, Wq, bq, Wk, bk, Wv, bv, Wo, bo, ffn_W1, ffn_b1,
           ffn_W2, ffn_b2, n1_g, n1_b, n2_g, n2_b, n3_g, n3_b):
    r = lambda a: a.reshape(1, -1)

    src = edge_index[0]
    dst = edge_index[1]
    srcp = (src // NPG) * P + (src % NPG)
    dstp = (dst // NPG) * P + (dst % NPG)
    half_mask = (dstp >= ACC_H).astype(jnp.int32)
    c0 = jnp.cumsum(1 - half_mask)
    c1 = jnp.cumsum(half_mask)
    k0 = c0[E - 1]
    rank = jnp.where(half_mask == 0, c0 - 1, c1 - 1)
    wkr = rank % NSUB
    slot = rank // NSUB
    flat = (half_mask * NSUB + wkr) * ECAP + slot
    total = NSC * NSUB * ECAP
    src_f = jnp.zeros((total,), jnp.int32).at[flat].set(srcp)
    dloc = dstp - half_mask * ACC_H
    dst_f = (ACC_H + (jnp.arange(total, dtype=jnp.int32) % P)) \
        .at[flat].set(dloc)
    cnt0 = (k0 - jnp.arange(NSUB, dtype=jnp.int32) + NSUB - 1) // NSUB
    cnt1 = (E - k0 - jnp.arange(NSUB, dtype=jnp.int32) + NSUB - 1) // NSUB
    cnt = jnp.stack([cnt0, cnt1])
    chn2 = jnp.maximum(2, ((cnt + EC - 1) // EC + 1) // 2 * 2)
    chn = jnp.zeros((NSC, NSUB, 16), jnp.int32).at[:, :, 0].set(chn2)
    src_i = src_f.reshape(NSC, NSUB, ECAP)
    dst_i = dst_f.reshape(NSC, NSUB, 2 * CHCAP, 128)
    zrows = jnp.zeros((ZR, C), jnp.float32)

    x_tp = (jnp.zeros((B, P, DM), jnp.float32)
            .at[:, :NPG, :].set(x_t.reshape(B, NPG, DM))
            .reshape(NP, DM))
    t_f = t.astype(jnp.float32).reshape(B, 1)

    temb = _temb(t_f, time_W1, r(time_b1), time_W2, r(time_b2))
    h = _inproj(x_tp, temb.reshape(B // G, G, DM), in_W, r(in_b))
    for l in range(L):
        agg2 = _sc_agg(h, src_i, dst_i, zrows, chn)
        h = _layer(gin_eps[l:l + 1], h, agg2,
                   gin_W1[l], r(gin_b1[l]), gin_W2[l], r(gin_b2[l]),
                   Wq[l], r(bq[l]), Wk[l], r(bk[l]), Wv[l], r(bv[l]),
                   Wo[l], r(bo[l]),
                   ffn_W1[l], r(ffn_b1[l]), ffn_W2[l], r(ffn_b2[l]),
                   r(n1_g[l]), r(n1_b[l]), r(n2_g[l]), r(n2_b[l]),
                   r(n3_g[l]), r(n3_b[l]))
    outp = _outproj(h, out_W, r(out_b))
    return outp.reshape(B, P, C)[:, :NPG, :].reshape(N, C)


# speedup vs baseline: 1.3759x; 1.2710x over previous
import functools
import math

import jax
import jax.numpy as jnp
from jax import lax
from jax.experimental import pallas as pl
from jax.experimental.pallas import tpu as pltpu
from jax.experimental.pallas import tpu_sc as plsc

N = 10000
B = 100
E = 320000
C = 128
DM = 256
L = 8
H = 4
DH = C // H
NPG = N // B
P = 128
NP = B * P
NSC = 2
NSUB = 16
NW = NSC * NSUB
EC = 128
KB = 16
CH = 80
NBLK = CH // KB
EPAD = NW * EC * CH
ACC_R = NP + P
ZR = ACC_R // NSUB
WR = NP // NSUB
G = 10
RG = G * P


def _ln(x, g, b):
    mu = jnp.mean(x, axis=-1, keepdims=True)
    xc = x - mu
    var = jnp.mean(xc * xc, axis=-1, keepdims=True)
    return xc * lax.rsqrt(var + 1e-5) * g + b


def _dot(a, b):
    return jnp.dot(a, b, preferred_element_type=jnp.float32)


def _temb_body(t_ref, w1, b1, w2, b2, out_ref):
    tf = t_ref[...]
    io = lax.broadcasted_iota(jnp.int32, (1, C // 2), 1).astype(jnp.float32)
    freqs = jnp.exp(io * (-math.log(10000.0) / (C // 2)))
    a = tf * freqs
    temb = jnp.concatenate([jnp.sin(a), jnp.cos(a)], axis=1)
    hh = jnp.maximum(_dot(temb, w1[...]) + b1[...], 0.0)
    out_ref[...] = _dot(hh, w2[...]) + b2[...]


def _temb(t_f, w1, b1, w2, b2):
    return pl.pallas_call(
        _temb_body,
        out_shape=jax.ShapeDtypeStruct((B, DM), jnp.float32),
    )(t_f, w1, b1, w2, b2)


def _inproj_body(xt_ref, te_ref, w_ref, b_ref, out_ref):
    rows = [xt_ref[j * P:(j + 1) * P, :] + te_ref[0, j] for j in range(G)]
    xin = jnp.concatenate(rows, axis=0)
    out_ref[...] = _dot(xin, w_ref[...]) + b_ref[...]


def _inproj(x_tp, temb, w, b):
    return pl.pallas_call(
        _inproj_body,
        grid=(B // G,),
        in_specs=[
            pl.BlockSpec((RG, DM), lambda i: (i, 0)),
            pl.BlockSpec((1, G, DM), lambda i: (i, 0, 0)),
            pl.BlockSpec((DM, C), lambda i: (0, 0)),
            pl.BlockSpec((1, C), lambda i: (0, 0)),
        ],
        out_specs=pl.BlockSpec((RG, C), lambda i: (i, 0)),
        out_shape=jax.ShapeDtypeStruct((NP, C), jnp.float32),
    )(x_tp, temb, w, b)


def _layer_body(eps_ref, x_ref, agg_ref,
                gw1, gb1, gw2, gb2,
                wq, bq, wk, bk, wv, bv, wo, bo,
                fw1, fb1, fw2, fb2,
                g1, b1, g2, b2, g3, b3, out_ref):
    x = x_ref[...]
    agg = agg_ref[0] + agg_ref[1]
    e1 = 1.0 + eps_ref[0]
    z = e1 * x + agg
    hg = jnp.maximum(_dot(z, gw1[...]) + gb1[...], 0.0)
    hg = _dot(hg, gw2[...]) + gb2[...]
    hg = _ln(hg + x, g1[...], b1[...])

    q = _dot(x, wq[...]) + bq[...]
    k = _dot(x, wk[...]) + bk[...]
    v = _dot(x, wv[...]) + bv[...]
    lane = lax.broadcasted_iota(jnp.int32, (P, C), 1)
    hmasks = [(lane // DH) == h for h in range(H)]
    smask = lax.broadcasted_iota(jnp.int32, (P, P), 1) >= NPG
    scale = 1.0 / math.sqrt(DH)
    outs = []
    for j in range(G):
        qj = q[j * P:(j + 1) * P, :]
        kj = k[j * P:(j + 1) * P, :]
        vj = v[j * P:(j + 1) * P, :]
        oj = None
        for h in range(H):
            kh = jnp.where(hmasks[h], kj, 0.0)
            s = lax.dot_general(qj, kh, (((1,), (1,)), ((), ())),
                                preferred_element_type=jnp.float32) * scale
            s = jnp.where(smask, -1e30, s)
            m = jnp.max(s, axis=-1, keepdims=True)
            p_ = jnp.exp(s - m)
            p_ = p_ / jnp.sum(p_, axis=-1, keepdims=True)
            vh = jnp.where(hmasks[h], vj, 0.0)
            contrib = _dot(p_, vh)
            oj = contrib if oj is None else oj + contrib
        outs.append(oj)
    o = jnp.concatenate(outs, axis=0)
    ha = _dot(o, wo[...]) + bo[...]
    ha = _ln(ha + x, g2[...], b2[...])

    out = hg + ha
    hid = jnp.maximum(_dot(out, fw1[...]) + fb1[...], 0.0)
    out = out + _dot(hid, fw2[...]) + fb2[...]
    out_ref[...] = _ln(out, g3[...], b3[...])


def _layer(eps, x, agg2, gw1, gb1, gw2, gb2, wq, bq, wk, bk, wv, bv, wo, bo,
           fw1, fb1, fw2, fb2, g1, b1, g2, b2, g3, b3):
    full = lambda r, c: pl.BlockSpec((r, c), lambda i: (0, 0))
    return pl.pallas_call(
        _layer_body,
        grid=(B // G,),
        in_specs=[
            pl.BlockSpec(memory_space=pltpu.SMEM),
            pl.BlockSpec((RG, C), lambda i: (i, 0)),
            pl.BlockSpec((NSC, RG, C), lambda i: (0, i, 0)),
            full(C, C), full(1, C), full(C, C), full(1, C),
            full(C, C), full(1, C), full(C, C), full(1, C),
            full(C, C), full(1, C), full(C, C), full(1, C),
            full(C, 2 * C), full(1, 2 * C), full(2 * C, C), full(1, C),
            full(1, C), full(1, C), full(1, C), full(1, C),
            full(1, C), full(1, C),
        ],
        out_specs=pl.BlockSpec((RG, C), lambda i: (i, 0)),
        out_shape=jax.ShapeDtypeStruct((NP, C), jnp.float32),
    )(eps, x, agg2, gw1, gb1, gw2, gb2, wq, bq, wk, bk, wv, bv, wo, bo,
      fw1, fb1, fw2, fb2, g1, b1, g2, b2, g3, b3)


def _outproj_body(x_ref, w_ref, b_ref, out_ref):
    out_ref[...] = _dot(x_ref[...], w_ref[...]) + b_ref[...]


def _outproj(x, w, b):
    return pl.pallas_call(
        _outproj_body,
        grid=(B // G,),
        in_specs=[
            pl.BlockSpec((RG, C), lambda i: (i, 0)),
            pl.BlockSpec((C, C), lambda i: (0, 0)),
            pl.BlockSpec((1, C), lambda i: (0, 0)),
        ],
        out_specs=pl.BlockSpec((RG, C), lambda i: (i, 0)),
        out_shape=jax.ShapeDtypeStruct((NP, C), jnp.float32),
    )(x, w, b)


def _sc_agg(x_p, src_i, dst_i, zrows):
    mesh = plsc.VectorSubcoreMesh(core_axis_name="c", subcore_axis_name="s")

    @functools.partial(
        pl.kernel,
        out_type=jax.ShapeDtypeStruct((NSC, NP, C), jnp.float32),
        mesh=mesh,
        scratch_types=[
            pltpu.VMEM((KB, EC), jnp.int32),
            pltpu.VMEM((KB, EC), jnp.int32),
            pltpu.VMEM((EC, C), jnp.float32),
            pltpu.VMEM_SHARED((ACC_R, C), jnp.float32),
            pltpu.SemaphoreType.DMA,
        ],
    )
    def agg_kernel(x_hbm, src_hbm, dst_hbm, z_hbm, out_hbm,
                   src_v, dst_v, gbuf, acc, sem):
        cid = lax.axis_index("c")
        sid = lax.axis_index("s")
        wid = cid * NSUB + sid
        pltpu.sync_copy(z_hbm, acc.at[pl.ds(sid * ZR, ZR)])
        plsc.subcore_barrier()

        def blk(bi, carry):
            pltpu.sync_copy(src_hbm.at[wid, pl.ds(bi * KB, KB)], src_v)
            pltpu.sync_copy(dst_hbm.at[wid, pl.ds(bi * KB, KB)], dst_v)

            def body(r, c2):
                pltpu.async_copy(x_hbm.at[src_v.at[r]], gbuf, sem).wait()
                pltpu.sync_copy(gbuf, acc.at[dst_v.at[r]], add=True)
                return c2

            lax.fori_loop(0, KB, body, 0)
            return carry

        lax.fori_loop(0, NBLK, blk, 0)
        plsc.subcore_barrier()
        pltpu.sync_copy(acc.at[pl.ds(sid * WR, WR)],
                        out_hbm.at[cid, pl.ds(sid * WR, WR)])

    return agg_kernel(x_p, src_i, dst_i, zrows)


def kernel(x_t, edge_index, batch, num_nodes, t, time_W1, time_b1, time_W2,
           time_b2, in_W, in_b, out_W, out_b, gin_eps, gin_W1, gin_b1,
           gin_W2, gin_b2, Wq, bq, Wk, bk, Wv, bv, Wo, bo, ffn_W1, ffn_b1,
           ffn_W2, ffn_b2, n1_g, n1_b, n2_g, n2_b, n3_g, n3_b):
    r = lambda a: a.reshape(1, -1)

    src = edge_index[0]
    dst = edge_index[1]
    srcp = (src // NPG) * P + (src % NPG)
    dstp = (dst // NPG) * P + (dst % NPG)
    pad = EPAD - E
    srcp = jnp.concatenate([srcp, jnp.zeros((pad,), jnp.int32)])
    dstp = jnp.concatenate(
        [dstp, NP + (jnp.arange(pad, dtype=jnp.int32) % P)])
    src_i = srcp.reshape(NW, CH, EC)
    dst_i = dstp.reshape(NW, CH, EC)
    zrows = jnp.zeros((ZR, C), jnp.float32)

    x_tp = (jnp.zeros((B, P, DM), jnp.float32)
            .at[:, :NPG, :].set(x_t.reshape(B, NPG, DM))
            .reshape(NP, DM))
    t_f = t.astype(jnp.float32).reshape(B, 1)

    temb = _temb(t_f, time_W1, r(time_b1), time_W2, r(time_b2))
    h = _inproj(x_tp, temb.reshape(B // G, G, DM), in_W, r(in_b))
    for l in range(L):
        agg2 = _sc_agg(h, src_i, dst_i, zrows)
        h = _layer(gin_eps[l:l + 1], h, agg2,
                   gin_W1[l], r(gin_b1[l]), gin_W2[l], r(gin_b2[l]),
                   Wq[l], r(bq[l]), Wk[l], r(bk[l]), Wv[l], r(bv[l]),
                   Wo[l], r(bo[l]),
                   ffn_W1[l], r(ffn_b1[l]), ffn_W2[l], r(ffn_b2[l]),
                   r(n1_g[l]), r(n1_b[l]), r(n2_g[l]), r(n2_b[l]),
                   r(n3_g[l]), r(n3_b[l]))
    outp = _outproj(h, out_W, r(out_b))
    return outp.reshape(B, P, C)[:, :NPG, :].reshape(N, C)


# speedup vs baseline: 1.3862x; 1.0075x over previous
import functools
import math

import jax
import jax.numpy as jnp
from jax import lax
from jax.experimental import pallas as pl
from jax.experimental.pallas import tpu as pltpu
from jax.experimental.pallas import tpu_sc as plsc

N = 10000
B = 100
E = 320000
C = 128
DM = 256
L = 8
H = 4
DH = C // H
NPG = N // B
P = 128
NP = B * P
NSC = 2
NSUB = 16
NW = NSC * NSUB
EC = 128
KB = 16
CH = 80
NBLK = CH // KB
EPAD = NW * EC * CH
ACC_R = NP + P
ZR = ACC_R // NSUB
WR = NP // NSUB
G = 10
RG = G * P


def _ln(x, g, b):
    mu = jnp.mean(x, axis=-1, keepdims=True)
    xc = x - mu
    var = jnp.mean(xc * xc, axis=-1, keepdims=True)
    return xc * lax.rsqrt(var + 1e-5) * g + b


def _dot(a, b):
    return jnp.dot(a, b, preferred_element_type=jnp.float32)


def _temb_body(t_ref, w1, b1, w2, b2, out_ref):
    tf = t_ref[...]
    io = lax.broadcasted_iota(jnp.int32, (1, C // 2), 1).astype(jnp.float32)
    freqs = jnp.exp(io * (-math.log(10000.0) / (C // 2)))
    a = tf * freqs
    temb = jnp.concatenate([jnp.sin(a), jnp.cos(a)], axis=1)
    hh = jnp.maximum(_dot(temb, w1[...]) + b1[...], 0.0)
    out_ref[...] = _dot(hh, w2[...]) + b2[...]


def _temb(t_f, w1, b1, w2, b2):
    return pl.pallas_call(
        _temb_body,
        out_shape=jax.ShapeDtypeStruct((B, DM), jnp.float32),
    )(t_f, w1, b1, w2, b2)


def _inproj_body(xt_ref, te_ref, w_ref, b_ref, out_ref):
    rows = [xt_ref[j * P:(j + 1) * P, :] + te_ref[0, j] for j in range(G)]
    xin = jnp.concatenate(rows, axis=0)
    out_ref[...] = _dot(xin, w_ref[...]) + b_ref[...]


def _inproj(x_tp, temb, w, b):
    return pl.pallas_call(
        _inproj_body,
        grid=(B // G,),
        in_specs=[
            pl.BlockSpec((RG, DM), lambda i: (i, 0)),
            pl.BlockSpec((1, G, DM), lambda i: (i, 0, 0)),
            pl.BlockSpec((DM, C), lambda i: (0, 0)),
            pl.BlockSpec((1, C), lambda i: (0, 0)),
        ],
        out_specs=pl.BlockSpec((RG, C), lambda i: (i, 0)),
        out_shape=jax.ShapeDtypeStruct((NP, C), jnp.float32),
    )(x_tp, temb, w, b)


def _layer_body(eps_ref, x_ref, agg_ref,
                gw1, gb1, gw2, gb2,
                wq, bq, wk, bk, wv, bv, wo, bo,
                fw1, fb1, fw2, fb2,
                g1, b1, g2, b2, g3, b3, out_ref):
    x = x_ref[...]
    agg = agg_ref[0] + agg_ref[1]
    e1 = 1.0 + eps_ref[0]
    z = e1 * x + agg
    hg = jnp.maximum(_dot(z, gw1[...]) + gb1[...], 0.0)
    hg = _dot(hg, gw2[...]) + gb2[...]
    hg = _ln(hg + x, g1[...], b1[...])

    q = _dot(x, wq[...]) + bq[...]
    k = _dot(x, wk[...]) + bk[...]
    v = _dot(x, wv[...]) + bv[...]
    lane = lax.broadcasted_iota(jnp.int32, (P, C), 1)
    hmasks = [(lane // DH) == h for h in range(H)]
    smask = lax.broadcasted_iota(jnp.int32, (P, P), 1) >= NPG
    scale = 1.0 / math.sqrt(DH)
    outs = []
    for j in range(G):
        qj = q[j * P:(j + 1) * P, :]
        kj = k[j * P:(j + 1) * P, :]
        vj = v[j * P:(j + 1) * P, :]
        oj = None
        for h in range(H):
            kh = jnp.where(hmasks[h], kj, 0.0)
            s = lax.dot_general(qj, kh, (((1,), (1,)), ((), ())),
                                preferred_element_type=jnp.float32) * scale
            s = jnp.where(smask, -1e30, s)
            m = jnp.max(s, axis=-1, keepdims=True)
            p_ = jnp.exp(s - m)
            p_ = p_ / jnp.sum(p_, axis=-1, keepdims=True)
            vh = jnp.where(hmasks[h], vj, 0.0)
            contrib = _dot(p_, vh)
            oj = contrib if oj is None else oj + contrib
        outs.append(oj)
    o = jnp.concatenate(outs, axis=0)
    ha = _dot(o, wo[...]) + bo[...]
    ha = _ln(ha + x, g2[...], b2[...])

    out = hg + ha
    hid = jnp.maximum(_dot(out, fw1[...]) + fb1[...], 0.0)
    out = out + _dot(hid, fw2[...]) + fb2[...]
    out_ref[...] = _ln(out, g3[...], b3[...])


def _layer(eps, x, agg2, gw1, gb1, gw2, gb2, wq, bq, wk, bk, wv, bv, wo, bo,
           fw1, fb1, fw2, fb2, g1, b1, g2, b2, g3, b3):
    full = lambda r, c: pl.BlockSpec((r, c), lambda i: (0, 0))
    return pl.pallas_call(
        _layer_body,
        grid=(B // G,),
        in_specs=[
            pl.BlockSpec(memory_space=pltpu.SMEM),
            pl.BlockSpec((RG, C), lambda i: (i, 0)),
            pl.BlockSpec((NSC, RG, C), lambda i: (0, i, 0)),
            full(C, C), full(1, C), full(C, C), full(1, C),
            full(C, C), full(1, C), full(C, C), full(1, C),
            full(C, C), full(1, C), full(C, C), full(1, C),
            full(C, 2 * C), full(1, 2 * C), full(2 * C, C), full(1, C),
            full(1, C), full(1, C), full(1, C), full(1, C),
            full(1, C), full(1, C),
        ],
        out_specs=pl.BlockSpec((RG, C), lambda i: (i, 0)),
        out_shape=jax.ShapeDtypeStruct((NP, C), jnp.float32),
    )(eps, x, agg2, gw1, gb1, gw2, gb2, wq, bq, wk, bk, wv, bv, wo, bo,
      fw1, fb1, fw2, fb2, g1, b1, g2, b2, g3, b3)


def _outproj_body(x_ref, w_ref, b_ref, out_ref):
    out_ref[...] = _dot(x_ref[...], w_ref[...]) + b_ref[...]


def _outproj(x, w, b):
    return pl.pallas_call(
        _outproj_body,
        grid=(B // G,),
        in_specs=[
            pl.BlockSpec((RG, C), lambda i: (i, 0)),
            pl.BlockSpec((C, C), lambda i: (0, 0)),
            pl.BlockSpec((1, C), lambda i: (0, 0)),
        ],
        out_specs=pl.BlockSpec((RG, C), lambda i: (i, 0)),
        out_shape=jax.ShapeDtypeStruct((NP, C), jnp.float32),
    )(x, w, b)


def _sc_agg(x_p, src_i, dst_i, zrows):
    mesh = plsc.VectorSubcoreMesh(core_axis_name="c", subcore_axis_name="s")

    @functools.partial(
        pl.kernel,
        out_type=jax.ShapeDtypeStruct((NSC, NP, C), jnp.float32),
        mesh=mesh,
        scratch_types=[
            pltpu.VMEM((2 * KB, EC), jnp.int32),
            pltpu.VMEM((2 * KB, EC), jnp.int32),
            pltpu.VMEM((EC, C), jnp.float32),
            pltpu.VMEM_SHARED((ACC_R, C), jnp.float32),
            pltpu.SemaphoreType.DMA,
            pltpu.SemaphoreType.DMA,
        ],
    )
    def agg_kernel(x_hbm, src_hbm, dst_hbm, z_hbm, out_hbm,
                   src_v, dst_v, gbuf, acc, sem, isem):
        cid = lax.axis_index("c")
        sid = lax.axis_index("s")
        wid = cid * NSUB + sid

        def stage(bi):
            half = (bi % 2) * KB
            pltpu.async_copy(src_hbm.at[wid, pl.ds(bi * KB, KB)],
                             src_v.at[pl.ds(half, KB)], isem)
            pltpu.async_copy(dst_hbm.at[wid, pl.ds(bi * KB, KB)],
                             dst_v.at[pl.ds(half, KB)], isem)

        def stage_wait(bi):
            half = (bi % 2) * KB
            pltpu.make_async_copy(src_hbm.at[wid, pl.ds(bi * KB, KB)],
                                  src_v.at[pl.ds(half, KB)], isem).wait()
            pltpu.make_async_copy(dst_hbm.at[wid, pl.ds(bi * KB, KB)],
                                  dst_v.at[pl.ds(half, KB)], isem).wait()

        stage(0)
        pltpu.sync_copy(z_hbm, acc.at[pl.ds(sid * ZR, ZR)])
        plsc.subcore_barrier()

        def blk(bi, carry):
            stage_wait(bi)

            @pl.when(bi + 1 < NBLK)
            def _():
                stage(bi + 1)

            half = (bi % 2) * KB

            def body(r, c2):
                pltpu.async_copy(x_hbm.at[src_v.at[half + r]],
                                 gbuf, sem).wait()
                pltpu.sync_copy(gbuf, acc.at[dst_v.at[half + r]], add=True)
                return c2

            lax.fori_loop(0, KB, body, 0)
            return carry

        lax.fori_loop(0, NBLK, blk, 0)
        plsc.subcore_barrier()
        pltpu.sync_copy(acc.at[pl.ds(sid * WR, WR)],
                        out_hbm.at[cid, pl.ds(sid * WR, WR)])

    return agg_kernel(x_p, src_i, dst_i, zrows)


def kernel(x_t, edge_index, batch, num_nodes, t, time_W1, time_b1, time_W2,
           time_b2, in_W, in_b, out_W, out_b, gin_eps, gin_W1, gin_b1,
           gin_W2, gin_b2, Wq, bq, Wk, bk, Wv, bv, Wo, bo, ffn_W1, ffn_b1,
           ffn_W2, ffn_b2, n1_g, n1_b, n2_g, n2_b, n3_g, n3_b):
    r = lambda a: a.reshape(1, -1)

    src = edge_index[0]
    dst = edge_index[1]
    srcp = (src // NPG) * P + (src % NPG)
    dstp = (dst // NPG) * P + (dst % NPG)
    pad = EPAD - E
    srcp = jnp.concatenate([srcp, jnp.zeros((pad,), jnp.int32)])
    dstp = jnp.concatenate(
        [dstp, NP + (jnp.arange(pad, dtype=jnp.int32) % P)])
    src_i = srcp.reshape(NW, CH, EC)
    dst_i = dstp.reshape(NW, CH, EC)
    zrows = jnp.zeros((ZR, C), jnp.float32)

    x_tp = (jnp.zeros((B, P, DM), jnp.float32)
            .at[:, :NPG, :].set(x_t.reshape(B, NPG, DM))
            .reshape(NP, DM))
    t_f = t.astype(jnp.float32).reshape(B, 1)

    temb = _temb(t_f, time_W1, r(time_b1), time_W2, r(time_b2))
    h = _inproj(x_tp, temb.reshape(B // G, G, DM), in_W, r(in_b))
    for l in range(L):
        agg2 = _sc_agg(h, src_i, dst_i, zrows)
        h = _layer(gin_eps[l:l + 1], h, agg2,
                   gin_W1[l], r(gin_b1[l]), gin_W2[l], r(gin_b2[l]),
                   Wq[l], r(bq[l]), Wk[l], r(bk[l]), Wv[l], r(bv[l]),
                   Wo[l], r(bo[l]),
                   ffn_W1[l], r(ffn_b1[l]), ffn_W2[l], r(ffn_b2[l]),
                   r(n1_g[l]), r(n1_b[l]), r(n2_g[l]), r(n2_b[l]),
                   r(n3_g[l]), r(n3_b[l]))
    outp = _outproj(h, out_W, r(out_b))
    return outp.reshape(B, P, C)[:, :NPG, :].reshape(N, C)
